# baseline (speedup 1.0000x reference)
import functools

import jax
import jax.numpy as jnp
from jax import lax
from jax.experimental import pallas as pl
from jax.experimental.pallas import tpu as pltpu
from jax.experimental.pallas import tpu_sc as plsc

INPUT_DIM = 128
D2 = 2 * INPUT_DIM
BATCH = 16384
ASPECT_NUM = 100000
BROWS = -(-ASPECT_NUM // 128)
NC, NS, L = 2, 16, 16
NW = NC * NS
BPW = BATCH // NW
C = 64
NCHUNK = BPW // C


def _perm(v, pat):
    return lax.gather(
        v, pat[:, None],
        lax.GatherDimensionNumbers(offset_dims=(), collapsed_slice_dims=(0,),
                                   start_index_map=(0,)),
        slice_sizes=(1,), mode=lax.GatherScatterMode.PROMISE_IN_BOUNDS)


def _body(x_hbm, ids_hbm, w_hbm, b_hbm, out_hbm,
          idx_v, w_v, x_v, b0_v, b1_v, brow_v, out_v, sems, isem):
    wid = lax.axis_index("s") * NC + lax.axis_index("c")
    base = wid * BPW

    idx_copies = [pltpu.async_copy(ids_hbm.at[pl.ds(base + ci * C, C)],
                                   idx_v.at[ci], isem) for ci in range(NCHUNK)]
    for c in idx_copies:
        c.wait()
    for ci in range(NCHUNK):
        for i in range(C // L):
            r = lax.shift_right_logical(idx_v[ci, pl.ds(i * L, L)], 7)
            brow_v[0, ci, pl.ds(i * L, L)] = r
            brow_v[1, ci, pl.ds(i * L, L)] = r + BROWS

    iota = lax.iota(jnp.int32, L)
    pats = [iota // 2, 8 + iota // 2]
    lane_masks = [iota == j for j in range(L)]
    half_iota = iota // 2
    par_iota = iota % 2
    parity = par_iota == 1
    out_off = par_iota * 128 + half_iota

    def fire(ci, slot):
        w_c = pltpu.async_copy(w_hbm.at[idx_v.at[ci]], w_v.at[slot],
                               sems[0].at[slot])
        b0_c = pltpu.async_copy(b_hbm.at[brow_v.at[0, ci]],
                                b0_v.at[slot], sems[1].at[slot])
        b1_c = pltpu.async_copy(b_hbm.at[brow_v.at[1, ci]],
                                b1_v.at[slot], sems[2].at[slot])
        x_c = pltpu.async_copy(x_hbm.at[pl.ds(base + ci * C, C)], x_v.at[slot],
                               sems[3].at[slot])
        return (w_c, b0_c, b1_c, x_c)

    fire(0, 0)
    fire(1, 1)

    def chunk_pair(ci2, carry):
        for sub in range(2):
            ci = ci2 * 2 + sub
            pltpu.make_async_copy(w_hbm.at[idx_v.at[ci]],
                                  w_v.at[sub], sems[0].at[sub]).wait()
            pltpu.make_async_copy(b_hbm.at[brow_v.at[0, ci]],
                                  b0_v.at[sub], sems[1].at[sub]).wait()
            pltpu.make_async_copy(b_hbm.at[brow_v.at[1, ci]],
                                  b1_v.at[sub], sems[2].at[sub]).wait()
            pltpu.make_async_copy(x_hbm.at[pl.ds(base + ci * C, C)],
                                  x_v.at[sub], sems[3].at[sub]).wait()

            slot_splat = jnp.full((L,), sub, jnp.int32)

            def group_body(g, gcarry, ci=ci, sub=sub, slot_splat=slot_splat):
                rows8 = g * 8 + half_iota
                ids8 = plsc.load_gather(idx_v, [jnp.full((L,), ci, jnp.int32),
                                                rows8])
                bc = ids8 & 127
                b0v = plsc.load_gather(b0_v, [slot_splat, rows8, bc])
                b1v = plsc.load_gather(b1_v, [slot_splat, rows8, bc])
                outvec = jnp.where(parity, b1v, b0v)
                for rr in range(8):
                    r = g * 8 + rr
                    acc = jnp.zeros((L,), jnp.float32)
                    for j in range(D2 // L):
                        wv = w_v[sub, r, pl.ds(j * L, L)]
                        if j % 2 == 0:
                            xv = x_v[sub, r, pl.ds((j // 2) * L, L)]
                        xe = _perm(xv, pats[j % 2])
                        acc = acc + wv * xe
                    e0 = jnp.where(parity, 0.0, acc)
                    outvec = jnp.where(lane_masks[2 * rr],
                                       outvec + jnp.sum(e0), outvec)
                    outvec = jnp.where(lane_masks[2 * rr + 1],
                                       outvec + jnp.sum(acc - e0), outvec)
                q = base + ci * C + g * 8
                posbase = (q // 128) * 256 + (q % 128) - base * 2
                plsc.store_scatter(out_v, [posbase + out_off], outvec)
                return gcarry

            lax.fori_loop(0, C // 8, group_body, 0, unroll=2)

            @pl.when(ci2 < NCHUNK // 2 - 1)
            def _(ci=ci, sub=sub):
                fire(ci + 2, sub)

        return carry

    lax.fori_loop(0, NCHUNK // 2, chunk_pair, 0)

    pltpu.sync_copy(out_v, out_hbm.at[pl.ds(base * 2, BPW * 2)])


@jax.jit
def _run(X, ids, W_embs, b2):
    mesh = plsc.VectorSubcoreMesh(core_axis_name="c", subcore_axis_name="s",
                                  num_cores=NC, num_subcores=NS)
    f = functools.partial(
        pl.kernel,
        out_type=jax.ShapeDtypeStruct((BATCH * 2,), jnp.float32),
        mesh=mesh,
        compiler_params=pltpu.CompilerParams(needs_layout_passes=False),
        scratch_types=[
            pltpu.VMEM((NCHUNK, C), jnp.int32),
            pltpu.VMEM((2, C, D2), jnp.float32),
            pltpu.VMEM((2, C, INPUT_DIM), jnp.float32),
            pltpu.VMEM((2, C, 128), jnp.float32),
            pltpu.VMEM((2, C, 128), jnp.float32),
            pltpu.VMEM((2, NCHUNK, C), jnp.int32),
            pltpu.VMEM((BPW * 2,), jnp.float32),
            [pltpu.SemaphoreType.DMA((2,))] * 4,
            pltpu.SemaphoreType.DMA,
        ],
    )(_body)
    return f(X, ids, W_embs, b2)


def kernel(X, aspect_ids, W_embs, b_embs):
    pad = (-ASPECT_NUM) % 128
    b2 = jnp.pad(b_embs, ((0, pad), (0, 0))).T.reshape(2 * BROWS, 128)
    flat = _run(X, aspect_ids.astype(jnp.int32), W_embs, b2)
    return flat.reshape(BATCH // 128, 2, 128).transpose(0, 2, 1).reshape(BATCH, 2)


# speedup vs baseline: 1.0250x; 1.0250x over previous
import functools

import jax
import jax.numpy as jnp
from jax import lax
from jax.experimental import pallas as pl
from jax.experimental.pallas import tpu as pltpu
from jax.experimental.pallas import tpu_sc as plsc

INPUT_DIM = 128
D2 = 2 * INPUT_DIM
BATCH = 16384
ASPECT_NUM = 100000
BROWS = -(-ASPECT_NUM // 128)
NC, NS, L = 2, 16, 16
NW = NC * NS
BPW = BATCH // NW
C = 64
NCHUNK = BPW // C


def _perm(v, pat):
    return lax.gather(
        v, pat[:, None],
        lax.GatherDimensionNumbers(offset_dims=(), collapsed_slice_dims=(0,),
                                   start_index_map=(0,)),
        slice_sizes=(1,), mode=lax.GatherScatterMode.PROMISE_IN_BOUNDS)


def _body(x_hbm, ids_hbm, w_hbm, b_hbm, out_hbm,
          idx_v, w_v, x_v, b0_v, b1_v, brow_v, out_v, sems, isem):
    wid = lax.axis_index("s") * NC + lax.axis_index("c")
    base = wid * BPW

    idx_copies = [pltpu.async_copy(ids_hbm.at[pl.ds(base + ci * C, C)],
                                   idx_v.at[ci], isem) for ci in range(NCHUNK)]
    for c in idx_copies:
        c.wait()
    for ci in range(NCHUNK):
        for i in range(C // L):
            r = lax.shift_right_logical(idx_v[ci, pl.ds(i * L, L)], 7)
            brow_v[0, ci, pl.ds(i * L, L)] = r
            brow_v[1, ci, pl.ds(i * L, L)] = r + BROWS

    iota = lax.iota(jnp.int32, L)
    pats = [iota // 2, 8 + iota // 2]
    lane_masks = [iota == j for j in range(L)]
    half_iota = iota // 2
    par_iota = iota % 2
    parity = par_iota == 1
    out_off = par_iota * 128 + half_iota

    def fire(ci, slot):
        w_c = pltpu.async_copy(w_hbm.at[idx_v.at[ci]], w_v.at[slot],
                               sems[0].at[slot])
        b0_c = pltpu.async_copy(b_hbm.at[brow_v.at[0, ci]],
                                b0_v.at[slot], sems[1].at[slot])
        b1_c = pltpu.async_copy(b_hbm.at[brow_v.at[1, ci]],
                                b1_v.at[slot], sems[2].at[slot])
        x_c = pltpu.async_copy(x_hbm.at[pl.ds(base + ci * C, C)], x_v.at[slot],
                               sems[3].at[slot])
        return (w_c, b0_c, b1_c, x_c)

    fire(0, 0)
    fire(1, 1)

    def chunk_pair(ci2, carry):
        for sub in range(2):
            ci = ci2 * 2 + sub
            pltpu.make_async_copy(w_hbm.at[idx_v.at[ci]],
                                  w_v.at[sub], sems[0].at[sub]).wait()
            pltpu.make_async_copy(b_hbm.at[brow_v.at[0, ci]],
                                  b0_v.at[sub], sems[1].at[sub]).wait()
            pltpu.make_async_copy(b_hbm.at[brow_v.at[1, ci]],
                                  b1_v.at[sub], sems[2].at[sub]).wait()
            pltpu.make_async_copy(x_hbm.at[pl.ds(base + ci * C, C)],
                                  x_v.at[sub], sems[3].at[sub]).wait()

            slot_splat = jnp.full((L,), sub, jnp.int32)

            def group_body(g, gcarry, ci=ci, sub=sub, slot_splat=slot_splat):
                rows8 = g * 8 + half_iota
                ids8 = plsc.load_gather(idx_v, [jnp.full((L,), ci, jnp.int32),
                                                rows8])
                bc = ids8 & 127
                b0v = plsc.load_gather(b0_v, [slot_splat, rows8, bc])
                b1v = plsc.load_gather(b1_v, [slot_splat, rows8, bc])
                outvec = jnp.where(parity, b1v, b0v)
                for rr in range(8):
                    r = g * 8 + rr
                    acc = jnp.zeros((L,), jnp.float32)
                    for j in range(D2 // L):
                        wv = w_v[sub, r, pl.ds(j * L, L)]
                        if j % 2 == 0:
                            xv = x_v[sub, r, pl.ds((j // 2) * L, L)]
                        xe = _perm(xv, pats[j % 2])
                        acc = acc + wv * xe
                    e0 = jnp.where(parity, 0.0, acc)
                    outvec = jnp.where(lane_masks[2 * rr],
                                       outvec + jnp.sum(e0), outvec)
                    outvec = jnp.where(lane_masks[2 * rr + 1],
                                       outvec + jnp.sum(acc - e0), outvec)
                q = base + ci * C + g * 8
                posbase = (q // 128) * 256 + (q % 128) - base * 2
                plsc.store_scatter(out_v, [posbase + out_off], outvec)
                return gcarry

            lax.fori_loop(0, C // 8, group_body, 0)

            @pl.when(ci2 < NCHUNK // 2 - 1)
            def _(ci=ci, sub=sub):
                fire(ci + 2, sub)

        return carry

    lax.fori_loop(0, NCHUNK // 2, chunk_pair, 0)

    pltpu.sync_copy(out_v, out_hbm.at[pl.ds(base * 2, BPW * 2)])


@jax.jit
def _run(X, ids, W_embs, b2):
    mesh = plsc.VectorSubcoreMesh(core_axis_name="c", subcore_axis_name="s",
                                  num_cores=NC, num_subcores=NS)
    f = functools.partial(
        pl.kernel,
        out_type=jax.ShapeDtypeStruct((BATCH * 2,), jnp.float32),
        mesh=mesh,
        compiler_params=pltpu.CompilerParams(needs_layout_passes=False),
        scratch_types=[
            pltpu.VMEM((NCHUNK, C), jnp.int32),
            pltpu.VMEM((2, C, D2), jnp.float32),
            pltpu.VMEM((2, C, INPUT_DIM), jnp.float32),
            pltpu.VMEM((2, C, 128), jnp.float32),
            pltpu.VMEM((2, C, 128), jnp.float32),
            pltpu.VMEM((2, NCHUNK, C), jnp.int32),
            pltpu.VMEM((BPW * 2,), jnp.float32),
            [pltpu.SemaphoreType.DMA((2,))] * 4,
            pltpu.SemaphoreType.DMA,
        ],
    )(_body)
    return f(X, ids, W_embs, b2)


def kernel(X, aspect_ids, W_embs, b_embs):
    pad = (-ASPECT_NUM) % 128
    b2 = jnp.pad(b_embs, ((0, pad), (0, 0))).T.reshape(2 * BROWS, 128)
    flat = _run(X, aspect_ids.astype(jnp.int32), W_embs, b2)
    return flat.reshape(BATCH // 128, 2, 128).transpose(0, 2, 1).reshape(BATCH, 2)
